# baseline (speedup 1.0000x reference)
import functools

import jax
import jax.numpy as jnp
from jax import lax
from jax.experimental import pallas as pl
from jax.experimental.pallas import tpu as pltpu
from jax.experimental.pallas import tpu_sc as plsc

_N = 10000
_E = 320000
_D = 128
_NC = 2
_NS = 16
_NW = _NC * _NS
_EPW = _E // _NW
_K = 80
_NCHUNK = _EPW // _K
_RPT = 624
_TAIL = _N - _RPT * _NS
_ZR = 208
_LANES = 16


def _sc_body(src_hbm, dst_hbm, w_hbm, x_hbm, out_hbm,
             ib0, ib1, wb0, wb1, db0, db1,
             rin0, rin1, rout0, rout1, aggr_sh,
             is0, is1, gat0, gat1, scat0, scat1):
    c = lax.axis_index("c")
    s = lax.axis_index("s")
    wid = c * _NS + s
    ib = (ib0, ib1)
    wb = (wb0, wb1)
    db = (db0, db1)
    isem = (is0, is1)
    rin = (rin0, rin1)
    rout = (rout0, rout1)
    gat = (gat0, gat1)
    scat = (scat0, scat1)

    zeros16 = jnp.zeros((_LANES,), jnp.float32)

    def zrow(r, carry):
        for j in range(_D // _LANES):
            rout0[r, pl.ds(j * _LANES, _LANES)] = zeros16
        return carry

    lax.fori_loop(0, _K, zrow, 0)
    for p in range(_RPT // _K):
        pltpu.sync_copy(rout0, aggr_sh.at[pl.ds(s * _RPT + p * _K, _K)])
    rem = _RPT - (_RPT // _K) * _K
    if rem:
        pltpu.sync_copy(rout0.at[pl.ds(0, rem)],
                        aggr_sh.at[pl.ds(s * _RPT + (_RPT // _K) * _K, rem)])

    @pl.when(s == 0)
    def _zero_tail():
        pltpu.sync_copy(rout0.at[pl.ds(0, _TAIL)], aggr_sh.at[pl.ds(_RPT * _NS, _TAIL)])

    plsc.subcore_barrier()

    def idx_start(g, q):
        pltpu.async_copy(src_hbm.at[wid, g], ib[q].at[pl.ds(0, 1)], isem[q])
        pltpu.async_copy(dst_hbm.at[wid, g], ib[q].at[pl.ds(1, 1)], isem[q])
        pltpu.async_copy(w_hbm.at[wid, g], wb[q], isem[q])

    def idx_wait(g, q):
        pltpu.make_async_copy(src_hbm.at[wid, g], ib[q].at[pl.ds(0, 1)], isem[q]).wait()
        pltpu.make_async_copy(dst_hbm.at[wid, g], ib[q].at[pl.ds(1, 1)], isem[q]).wait()
        pltpu.make_async_copy(w_hbm.at[wid, g], wb[q], isem[q]).wait()

    def gather_start(g, b):
        pltpu.async_copy(x_hbm.at[ib[b].at[0]], rin[b], gat[b])

    def gather_wait(g, b):
        pltpu.make_async_copy(x_hbm.at[ib[b].at[0]], rin[b], gat[b]).wait()

    def scatter_start(g, b):
        pltpu.async_copy(rout[b], aggr_sh.at[db[b].at[0]], scat[b], add=True)

    def scatter_wait(g, b):
        pltpu.make_async_copy(rout[b], aggr_sh.at[db[b].at[0]], scat[b]).wait()

    def copy_dst(b):
        for t in range(_K // _LANES):
            sl = pl.ds(t * _LANES, _LANES)
            db[b][0, sl] = ib[b][1, sl]

    def scale(g, b):
        def edge_block(eb, c2):
            wv = wb[b][0, pl.ds(eb * _LANES, _LANES)]
            for t in range(_LANES):
                w = wv[t]
                i = eb * _LANES + t
                for j in range(_D // _LANES):
                    sl = pl.ds(j * _LANES, _LANES)
                    rout[b][i, sl] = rin[b][i, sl] * w
            return c2

        lax.fori_loop(0, _K // _LANES, edge_block, 0)

    def step(g, b, first, start1, start2):
        if start1:
            idx_wait(g + 1, 1 - b)
            gather_start(g + 1, 1 - b)
        gather_wait(g, b)
        if not first:
            scatter_wait(g - 2, b)
        copy_dst(b)
        scale(g, b)
        scatter_start(g, b)
        if start2:
            idx_start(g + 2, b)
        return

    idx_start(0, 0)
    idx_start(1, 1)
    idx_wait(0, 0)
    gather_start(0, 0)
    step(0, 0, True, True, True)
    step(1, 1, True, True, True)
    def pair(p, carry):
        for u in range(2):
            g = 2 * p + u
            step(g, u, False, True, True)
        return carry

    lax.fori_loop(1, 1 + (_NCHUNK - 5) // 2, pair, 0)
    for g in range(_NCHUNK - 3, _NCHUNK):
        step(g, g % 2, False, g + 1 < _NCHUNK, g + 2 < _NCHUNK)
    scatter_wait(_NCHUNK - 2, (_NCHUNK - 2) % 2)
    scatter_wait(_NCHUNK - 1, (_NCHUNK - 1) % 2)
    plsc.subcore_barrier()

    r0 = s * _RPT
    pltpu.sync_copy(aggr_sh.at[pl.ds(r0, _RPT)], out_hbm.at[c, pl.ds(r0, _RPT)])

    @pl.when(s == 0)
    def _write_tail():
        pltpu.sync_copy(aggr_sh.at[pl.ds(_RPT * _NS, _TAIL)],
                        out_hbm.at[c, pl.ds(_RPT * _NS, _TAIL)])


_sc_aggr = functools.partial(
    pl.kernel,
    mesh=plsc.VectorSubcoreMesh(core_axis_name="c", subcore_axis_name="s"),
    out_type=jax.ShapeDtypeStruct((_NC, _N, _D), jnp.float32),
    scratch_types=(
        [pltpu.VMEM((2, _K), jnp.int32)] * 2
        + [pltpu.VMEM((1, _K), jnp.float32)] * 2
        + [pltpu.VMEM((1, _K), jnp.int32)] * 2
        + [pltpu.VMEM((_K, _D), jnp.float32)] * 4
        + [pltpu.VMEM_SHARED((_N, _D), jnp.float32)]
        + [pltpu.SemaphoreType.DMA] * 6
    ),
)(_sc_body)


_BLK = 1000


def _tc_body(x_ref, w_ref, b_ref, ag_ref, o_ref):
    dense = lax.dot_general(
        x_ref[...], w_ref[...], (((1,), (1,)), ((), ())),
        preferred_element_type=jnp.float32)
    o_ref[...] = dense + b_ref[...] + ag_ref[0] + ag_ref[1]


def _tc_combine(x, wt, b2, aggr2):
    return pl.pallas_call(
        _tc_body,
        grid=(_N // _BLK,),
        in_specs=[
            pl.BlockSpec((_BLK, _D), lambda i: (i, 0)),
            pl.BlockSpec((_D, _D), lambda i: (0, 0)),
            pl.BlockSpec((1, _D), lambda i: (0, 0)),
            pl.BlockSpec((_NC, _BLK, _D), lambda i: (0, i, 0)),
        ],
        out_specs=pl.BlockSpec((_BLK, _D), lambda i: (i, 0)),
        out_shape=jax.ShapeDtypeStruct((_N, _D), jnp.float32),
    )(x, wt, b2, aggr2)


def kernel(x, edge_index, edge_weight, W, b):
    src = edge_index[0].astype(jnp.int32).reshape(_NW, _NCHUNK, 1, _K)
    dst = edge_index[1].astype(jnp.int32).reshape(_NW, _NCHUNK, 1, _K)
    ew = edge_weight.reshape(_NW, _NCHUNK, 1, _K)
    aggr2 = _sc_aggr(src, dst, ew, x)
    return _tc_combine(x, W, b.reshape(1, _D), aggr2)


# speedup vs baseline: 1.1081x; 1.1081x over previous
import functools

import jax
import jax.numpy as jnp
from jax import lax
from jax.experimental import pallas as pl
from jax.experimental.pallas import tpu as pltpu
from jax.experimental.pallas import tpu_sc as plsc

_N = 10000
_E = 320000
_D = 128
_NC = 2
_NS = 16
_NW = _NC * _NS
_EPW = _E // _NW
_K = 80
_NCHUNK = _EPW // _K
_RPT = 624
_TAIL = _N - _RPT * _NS
_ZR = 208
_LANES = 16


def _sc_body(src_hbm, dst_hbm, w_hbm, x_hbm, out_hbm,
             ib0, ib1, wb0, wb1, db0, db1,
             rin0, rin1, rout0, rout1, aggr_sh,
             is0, is1, gat0, gat1, scat0, scat1):
    c = lax.axis_index("c")
    s = lax.axis_index("s")
    wid = c * _NS + s
    ib = (ib0, ib1)
    wb = (wb0, wb1)
    db = (db0, db1)
    isem = (is0, is1)
    rin = (rin0, rin1)
    rout = (rout0, rout1)
    gat = (gat0, gat1)
    scat = (scat0, scat1)

    zeros16 = jnp.zeros((_LANES,), jnp.float32)

    def zrow(r, carry):
        for j in range(_D // _LANES):
            rout0[r, pl.ds(j * _LANES, _LANES)] = zeros16
        return carry

    lax.fori_loop(0, _K, zrow, 0)
    for p in range(_RPT // _K):
        pltpu.sync_copy(rout0, aggr_sh.at[pl.ds(s * _RPT + p * _K, _K)])
    rem = _RPT - (_RPT // _K) * _K
    if rem:
        pltpu.sync_copy(rout0.at[pl.ds(0, rem)],
                        aggr_sh.at[pl.ds(s * _RPT + (_RPT // _K) * _K, rem)])

    @pl.when(s == 0)
    def _zero_tail():
        pltpu.sync_copy(rout0.at[pl.ds(0, _TAIL)], aggr_sh.at[pl.ds(_RPT * _NS, _TAIL)])

    plsc.subcore_barrier()

    def idx_start(g, q):
        base = wid * _EPW + g * _K
        pltpu.async_copy(src_hbm.at[pl.ds(base, _K)], ib[q].at[0], isem[q])
        pltpu.async_copy(dst_hbm.at[pl.ds(base, _K)], ib[q].at[1], isem[q])
        pltpu.async_copy(w_hbm.at[pl.ds(base, _K)], wb[q].at[0], isem[q])

    def idx_wait(g, q):
        base = wid * _EPW + g * _K
        pltpu.make_async_copy(src_hbm.at[pl.ds(base, _K)], ib[q].at[0], isem[q]).wait()
        pltpu.make_async_copy(dst_hbm.at[pl.ds(base, _K)], ib[q].at[1], isem[q]).wait()
        pltpu.make_async_copy(w_hbm.at[pl.ds(base, _K)], wb[q].at[0], isem[q]).wait()

    def gather_start(g, b):
        pltpu.async_copy(x_hbm.at[ib[b].at[0]], rin[b], gat[b])

    def gather_wait(g, b):
        pltpu.make_async_copy(x_hbm.at[ib[b].at[0]], rin[b], gat[b]).wait()

    def scatter_start(g, b):
        pltpu.async_copy(rout[b], aggr_sh.at[db[b].at[0]], scat[b], add=True)

    def scatter_wait(g, b):
        pltpu.make_async_copy(rout[b], aggr_sh.at[db[b].at[0]], scat[b]).wait()

    def copy_dst(b):
        for t in range(_K // _LANES):
            sl = pl.ds(t * _LANES, _LANES)
            db[b][0, sl] = ib[b][1, sl]

    def scale(g, b):
        def edge_block(eb, c2):
            wv = wb[b][0, pl.ds(eb * _LANES, _LANES)]
            for t in range(_LANES):
                w = wv[t]
                i = eb * _LANES + t
                for j in range(_D // _LANES):
                    sl = pl.ds(j * _LANES, _LANES)
                    rout[b][i, sl] = rin[b][i, sl] * w
            return c2

        lax.fori_loop(0, _K // _LANES, edge_block, 0)

    def step(g, b, first, start1, start2):
        if start1:
            idx_wait(g + 1, 1 - b)
            gather_start(g + 1, 1 - b)
        gather_wait(g, b)
        if not first:
            scatter_wait(g - 2, b)
        copy_dst(b)
        scale(g, b)
        scatter_start(g, b)
        if start2:
            idx_start(g + 2, b)
        return

    idx_start(0, 0)
    idx_start(1, 1)
    idx_wait(0, 0)
    gather_start(0, 0)
    step(0, 0, True, True, True)
    step(1, 1, True, True, True)
    def pair(p, carry):
        for u in range(2):
            g = 2 * p + u
            step(g, u, False, True, True)
        return carry

    lax.fori_loop(1, 1 + (_NCHUNK - 5) // 2, pair, 0)
    for g in range(_NCHUNK - 3, _NCHUNK):
        step(g, g % 2, False, g + 1 < _NCHUNK, g + 2 < _NCHUNK)
    scatter_wait(_NCHUNK - 2, (_NCHUNK - 2) % 2)
    scatter_wait(_NCHUNK - 1, (_NCHUNK - 1) % 2)
    plsc.subcore_barrier()

    r0 = s * _RPT
    pltpu.sync_copy(aggr_sh.at[pl.ds(r0, _RPT)], out_hbm.at[c, pl.ds(r0, _RPT)])

    @pl.when(s == 0)
    def _write_tail():
        pltpu.sync_copy(aggr_sh.at[pl.ds(_RPT * _NS, _TAIL)],
                        out_hbm.at[c, pl.ds(_RPT * _NS, _TAIL)])


_sc_aggr = functools.partial(
    pl.kernel,
    mesh=plsc.VectorSubcoreMesh(core_axis_name="c", subcore_axis_name="s"),
    out_type=jax.ShapeDtypeStruct((_NC, _N, _D), jnp.float32),
    scratch_types=(
        [pltpu.VMEM((2, _K), jnp.int32)] * 2
        + [pltpu.VMEM((1, _K), jnp.float32)] * 2
        + [pltpu.VMEM((1, _K), jnp.int32)] * 2
        + [pltpu.VMEM((_K, _D), jnp.float32)] * 4
        + [pltpu.VMEM_SHARED((_N, _D), jnp.float32)]
        + [pltpu.SemaphoreType.DMA] * 6
    ),
)(_sc_body)


_BLK = 1000


def _tc_body(x_ref, w_ref, b_ref, ag_ref, o_ref):
    dense = lax.dot_general(
        x_ref[...], w_ref[...], (((1,), (1,)), ((), ())),
        preferred_element_type=jnp.float32)
    o_ref[...] = dense + b_ref[...] + ag_ref[0] + ag_ref[1]


def _tc_combine(x, wt, b2, aggr2):
    return pl.pallas_call(
        _tc_body,
        grid=(_N // _BLK,),
        in_specs=[
            pl.BlockSpec((_BLK, _D), lambda i: (i, 0)),
            pl.BlockSpec((_D, _D), lambda i: (0, 0)),
            pl.BlockSpec((1, _D), lambda i: (0, 0)),
            pl.BlockSpec((_NC, _BLK, _D), lambda i: (0, i, 0)),
        ],
        out_specs=pl.BlockSpec((_BLK, _D), lambda i: (i, 0)),
        out_shape=jax.ShapeDtypeStruct((_N, _D), jnp.float32),
    )(x, wt, b2, aggr2)


def kernel(x, edge_index, edge_weight, W, b):
    src = edge_index[0].astype(jnp.int32)
    dst = edge_index[1].astype(jnp.int32)
    aggr2 = _sc_aggr(src, dst, edge_weight, x)
    return _tc_combine(x, W, b.reshape(1, _D), aggr2)


# speedup vs baseline: 1.1118x; 1.0034x over previous
import functools

import jax
import jax.numpy as jnp
from jax import lax
from jax.experimental import pallas as pl
from jax.experimental.pallas import tpu as pltpu
from jax.experimental.pallas import tpu_sc as plsc

_N = 10000
_E = 320000
_D = 128
_NC = 2
_NS = 16
_NW = _NC * _NS
_EPW = _E // _NW
_K = 80
_NCHUNK = _EPW // _K
_RPT = 624
_TAIL = _N - _RPT * _NS
_ZR = 208
_LANES = 16


def _sc_body(src_hbm, dst_hbm, w_hbm, x_hbm, out_hbm,
             ib0, ib1, wb0, wb1, db0, db1,
             rin0, rin1, rout0, rout1, aggr_sh,
             is0, is1, gat0, gat1, scat0, scat1):
    c = lax.axis_index("c")
    s = lax.axis_index("s")
    wid = c * _NS + s
    ib = (ib0, ib1)
    wb = (wb0, wb1)
    db = (db0, db1)
    isem = (is0, is1)
    rin = (rin0, rin1)
    rout = (rout0, rout1)
    gat = (gat0, gat1)
    scat = (scat0, scat1)

    def idx_start(g, q):
        base = wid * _EPW + g * _K
        pltpu.async_copy(src_hbm.at[pl.ds(base, _K)], ib[q].at[0], isem[q])
        pltpu.async_copy(dst_hbm.at[pl.ds(base, _K)], ib[q].at[1], isem[q])
        pltpu.async_copy(w_hbm.at[pl.ds(base, _K)], wb[q].at[0], isem[q])

    def idx_wait(g, q):
        base = wid * _EPW + g * _K
        pltpu.make_async_copy(src_hbm.at[pl.ds(base, _K)], ib[q].at[0], isem[q]).wait()
        pltpu.make_async_copy(dst_hbm.at[pl.ds(base, _K)], ib[q].at[1], isem[q]).wait()
        pltpu.make_async_copy(w_hbm.at[pl.ds(base, _K)], wb[q].at[0], isem[q]).wait()

    def gather_start(g, b):
        pltpu.async_copy(x_hbm.at[ib[b].at[0]], rin[b], gat[b])

    def gather_wait(g, b):
        pltpu.make_async_copy(x_hbm.at[ib[b].at[0]], rin[b], gat[b]).wait()

    def scatter_start(g, b):
        pltpu.async_copy(rout[b], aggr_sh.at[db[b].at[0]], scat[b], add=True)

    def scatter_wait(g, b):
        pltpu.make_async_copy(rout[b], aggr_sh.at[db[b].at[0]], scat[b]).wait()

    def copy_dst(b):
        for t in range(_K // _LANES):
            sl = pl.ds(t * _LANES, _LANES)
            db[b][0, sl] = ib[b][1, sl]

    def scale(g, b):
        def edge_block(eb, c2):
            wv = wb[b][0, pl.ds(eb * _LANES, _LANES)]
            for t in range(_LANES):
                w = wv[t]
                i = eb * _LANES + t
                for j in range(_D // _LANES):
                    sl = pl.ds(j * _LANES, _LANES)
                    rout[b][i, sl] = rin[b][i, sl] * w
            return c2

        lax.fori_loop(0, _K // _LANES, edge_block, 0)

    def step(g, b, first, start1, start2):
        if start1:
            idx_wait(g + 1, 1 - b)
            gather_start(g + 1, 1 - b)
        gather_wait(g, b)
        if not first:
            scatter_wait(g - 2, b)
        copy_dst(b)
        scale(g, b)
        scatter_start(g, b)
        if start2:
            idx_start(g + 2, b)
        return

    idx_start(0, 0)
    idx_start(1, 1)
    zeros16 = jnp.zeros((_LANES,), jnp.float32)

    def zrow(r, carry):
        for j in range(_D // _LANES):
            rout0[r, pl.ds(j * _LANES, _LANES)] = zeros16
        return carry

    lax.fori_loop(0, _K, zrow, 0)
    for p in range(_RPT // _K):
        pltpu.sync_copy(rout0, aggr_sh.at[pl.ds(s * _RPT + p * _K, _K)])
    rem = _RPT - (_RPT // _K) * _K
    if rem:
        pltpu.sync_copy(rout0.at[pl.ds(0, rem)],
                        aggr_sh.at[pl.ds(s * _RPT + (_RPT // _K) * _K, rem)])

    @pl.when(s == 0)
    def _zero_tail():
        pltpu.sync_copy(rout0.at[pl.ds(0, _TAIL)], aggr_sh.at[pl.ds(_RPT * _NS, _TAIL)])

    idx_wait(0, 0)
    gather_start(0, 0)
    plsc.subcore_barrier()
    step(0, 0, True, True, True)
    step(1, 1, True, True, True)
    def pair(p, carry):
        for u in range(2):
            g = 2 * p + u
            step(g, u, False, True, True)
        return carry

    lax.fori_loop(1, 1 + (_NCHUNK - 5) // 2, pair, 0)
    for g in range(_NCHUNK - 3, _NCHUNK):
        step(g, g % 2, False, g + 1 < _NCHUNK, g + 2 < _NCHUNK)
    scatter_wait(_NCHUNK - 2, (_NCHUNK - 2) % 2)
    scatter_wait(_NCHUNK - 1, (_NCHUNK - 1) % 2)
    plsc.subcore_barrier()

    r0 = s * _RPT
    pltpu.sync_copy(aggr_sh.at[pl.ds(r0, _RPT)], out_hbm.at[c, pl.ds(r0, _RPT)])

    @pl.when(s == 0)
    def _write_tail():
        pltpu.sync_copy(aggr_sh.at[pl.ds(_RPT * _NS, _TAIL)],
                        out_hbm.at[c, pl.ds(_RPT * _NS, _TAIL)])


_sc_aggr = functools.partial(
    pl.kernel,
    mesh=plsc.VectorSubcoreMesh(core_axis_name="c", subcore_axis_name="s"),
    out_type=jax.ShapeDtypeStruct((_NC, _N, _D), jnp.float32),
    scratch_types=(
        [pltpu.VMEM((2, _K), jnp.int32)] * 2
        + [pltpu.VMEM((1, _K), jnp.float32)] * 2
        + [pltpu.VMEM((1, _K), jnp.int32)] * 2
        + [pltpu.VMEM((_K, _D), jnp.float32)] * 4
        + [pltpu.VMEM_SHARED((_N, _D), jnp.float32)]
        + [pltpu.SemaphoreType.DMA] * 6
    ),
)(_sc_body)


_BLK = 1000


def _tc_body(x_ref, w_ref, b_ref, ag_ref, o_ref):
    dense = lax.dot_general(
        x_ref[...], w_ref[...], (((1,), (1,)), ((), ())),
        preferred_element_type=jnp.float32)
    o_ref[...] = dense + b_ref[...] + ag_ref[0] + ag_ref[1]


def _tc_combine(x, wt, b2, aggr2):
    return pl.pallas_call(
        _tc_body,
        grid=(_N // _BLK,),
        in_specs=[
            pl.BlockSpec((_BLK, _D), lambda i: (i, 0)),
            pl.BlockSpec((_D, _D), lambda i: (0, 0)),
            pl.BlockSpec((1, _D), lambda i: (0, 0)),
            pl.BlockSpec((_NC, _BLK, _D), lambda i: (0, i, 0)),
        ],
        out_specs=pl.BlockSpec((_BLK, _D), lambda i: (i, 0)),
        out_shape=jax.ShapeDtypeStruct((_N, _D), jnp.float32),
    )(x, wt, b2, aggr2)


def kernel(x, edge_index, edge_weight, W, b):
    src = edge_index[0].astype(jnp.int32)
    dst = edge_index[1].astype(jnp.int32)
    aggr2 = _sc_aggr(src, dst, edge_weight, x)
    return _tc_combine(x, W, b.reshape(1, _D), aggr2)


# speedup vs baseline: 1.1525x; 1.0366x over previous
import functools

import jax
import jax.numpy as jnp
from jax import lax
from jax.experimental import pallas as pl
from jax.experimental.pallas import tpu as pltpu
from jax.experimental.pallas import tpu_sc as plsc

_N = 10000
_E = 320000
_D = 128
_NC = 2
_NS = 16
_NW = _NC * _NS
_EPW = _E // _NW
_K = 80
_NCHUNK = _EPW // _K
_RPT = 624
_TAIL = _N - _RPT * _NS
_ZR = 208
_LANES = 16


def _sc_body(ei_hbm, w_hbm, x_hbm, out_hbm,
             ib0, ib1, wb0, wb1, db0, db1,
             rin0, rin1, rout0, rout1, aggr_sh,
             is0, is1, gat0, gat1, scat0, scat1):
    c = lax.axis_index("c")
    s = lax.axis_index("s")
    wid = c * _NS + s
    ib = (ib0, ib1)
    wb = (wb0, wb1)
    db = (db0, db1)
    isem = (is0, is1)
    rin = (rin0, rin1)
    rout = (rout0, rout1)
    gat = (gat0, gat1)
    scat = (scat0, scat1)

    def idx_start(g, q):
        base = wid * _EPW + g * _K
        pltpu.async_copy(ei_hbm.at[pl.ds(base, _K)], ib[q].at[0], isem[q])
        pltpu.async_copy(ei_hbm.at[pl.ds(_E + base, _K)], ib[q].at[1], isem[q])
        pltpu.async_copy(w_hbm.at[pl.ds(base, _K)], wb[q].at[0], isem[q])

    def idx_wait(g, q):
        base = wid * _EPW + g * _K
        pltpu.make_async_copy(ei_hbm.at[pl.ds(base, _K)], ib[q].at[0], isem[q]).wait()
        pltpu.make_async_copy(ei_hbm.at[pl.ds(_E + base, _K)], ib[q].at[1], isem[q]).wait()
        pltpu.make_async_copy(w_hbm.at[pl.ds(base, _K)], wb[q].at[0], isem[q]).wait()

    def gather_start(g, b):
        pltpu.async_copy(x_hbm.at[ib[b].at[0]], rin[b], gat[b])

    def gather_wait(g, b):
        pltpu.make_async_copy(x_hbm.at[ib[b].at[0]], rin[b], gat[b]).wait()

    def scatter_start(g, b):
        pltpu.async_copy(rout[b], aggr_sh.at[db[b].at[0]], scat[b], add=True)

    def scatter_wait(g, b):
        pltpu.make_async_copy(rout[b], aggr_sh.at[db[b].at[0]], scat[b]).wait()

    def copy_dst(b):
        for t in range(_K // _LANES):
            sl = pl.ds(t * _LANES, _LANES)
            db[b][0, sl] = ib[b][1, sl]

    def scale(g, b):
        def edge_block(eb, c2):
            wv = wb[b][0, pl.ds(eb * _LANES, _LANES)]
            for t in range(_LANES):
                w = wv[t]
                i = eb * _LANES + t
                for j in range(_D // _LANES):
                    sl = pl.ds(j * _LANES, _LANES)
                    rout[b][i, sl] = rin[b][i, sl] * w
            return c2

        lax.fori_loop(0, _K // _LANES, edge_block, 0)

    def step(g, b, first, start1, start2):
        if start1:
            idx_wait(g + 1, 1 - b)
            gather_start(g + 1, 1 - b)
        gather_wait(g, b)
        if not first:
            scatter_wait(g - 2, b)
        copy_dst(b)
        scale(g, b)
        scatter_start(g, b)
        if start2:
            idx_start(g + 2, b)
        return

    idx_start(0, 0)
    idx_start(1, 1)
    zeros16 = jnp.zeros((_LANES,), jnp.float32)

    def zrow(r, carry):
        for j in range(_D // _LANES):
            rout0[r, pl.ds(j * _LANES, _LANES)] = zeros16
        return carry

    lax.fori_loop(0, _K, zrow, 0)
    for p in range(_RPT // _K):
        pltpu.sync_copy(rout0, aggr_sh.at[pl.ds(s * _RPT + p * _K, _K)])
    rem = _RPT - (_RPT // _K) * _K
    if rem:
        pltpu.sync_copy(rout0.at[pl.ds(0, rem)],
                        aggr_sh.at[pl.ds(s * _RPT + (_RPT // _K) * _K, rem)])

    @pl.when(s == 0)
    def _zero_tail():
        pltpu.sync_copy(rout0.at[pl.ds(0, _TAIL)], aggr_sh.at[pl.ds(_RPT * _NS, _TAIL)])

    idx_wait(0, 0)
    gather_start(0, 0)
    plsc.subcore_barrier()
    step(0, 0, True, True, True)
    step(1, 1, True, True, True)
    def pair(p, carry):
        for u in range(2):
            g = 2 * p + u
            step(g, u, False, True, True)
        return carry

    lax.fori_loop(1, 1 + (_NCHUNK - 5) // 2, pair, 0)
    for g in range(_NCHUNK - 3, _NCHUNK):
        step(g, g % 2, False, g + 1 < _NCHUNK, g + 2 < _NCHUNK)
    scatter_wait(_NCHUNK - 2, (_NCHUNK - 2) % 2)
    scatter_wait(_NCHUNK - 1, (_NCHUNK - 1) % 2)
    plsc.subcore_barrier()

    r0 = s * _RPT
    pltpu.sync_copy(aggr_sh.at[pl.ds(r0, _RPT)], out_hbm.at[c, pl.ds(r0, _RPT)])

    @pl.when(s == 0)
    def _write_tail():
        pltpu.sync_copy(aggr_sh.at[pl.ds(_RPT * _NS, _TAIL)],
                        out_hbm.at[c, pl.ds(_RPT * _NS, _TAIL)])


_sc_aggr = functools.partial(
    pl.kernel,
    mesh=plsc.VectorSubcoreMesh(core_axis_name="c", subcore_axis_name="s"),
    out_type=jax.ShapeDtypeStruct((_NC, _N, _D), jnp.float32),
    scratch_types=(
        [pltpu.VMEM((2, _K), jnp.int32)] * 2
        + [pltpu.VMEM((1, _K), jnp.float32)] * 2
        + [pltpu.VMEM((1, _K), jnp.int32)] * 2
        + [pltpu.VMEM((_K, _D), jnp.float32)] * 4
        + [pltpu.VMEM_SHARED((_N, _D), jnp.float32)]
        + [pltpu.SemaphoreType.DMA] * 6
    ),
)(_sc_body)


_BLK = 1000


def _tc_body(x_ref, w_ref, b_ref, ag_ref, o_ref):
    dense = lax.dot_general(
        x_ref[...], w_ref[...], (((1,), (1,)), ((), ())),
        preferred_element_type=jnp.float32)
    o_ref[...] = dense + b_ref[...] + ag_ref[0] + ag_ref[1]


def _tc_combine(x, wt, b2, aggr2):
    return pl.pallas_call(
        _tc_body,
        grid=(_N // _BLK,),
        in_specs=[
            pl.BlockSpec((_BLK, _D), lambda i: (i, 0)),
            pl.BlockSpec((_D, _D), lambda i: (0, 0)),
            pl.BlockSpec((1, _D), lambda i: (0, 0)),
            pl.BlockSpec((_NC, _BLK, _D), lambda i: (0, i, 0)),
        ],
        out_specs=pl.BlockSpec((_BLK, _D), lambda i: (i, 0)),
        out_shape=jax.ShapeDtypeStruct((_N, _D), jnp.float32),
    )(x, wt, b2, aggr2)


def kernel(x, edge_index, edge_weight, W, b):
    ei = edge_index.astype(jnp.int32).reshape(2 * _E)
    aggr2 = _sc_aggr(ei, edge_weight, x)
    return _tc_combine(x, W, b.reshape(1, _D), aggr2)
